# baseline (speedup 1.0000x reference)
import functools

import jax
import jax.numpy as jnp
from jax import lax
from jax.experimental import pallas as pl
from jax.experimental.pallas import tpu as pltpu
from jax.experimental.pallas import tpu_sc as plsc

_NSUB = 16
_F32 = jnp.float32


def _make_sc_segsum(n, e, w, k):
    ept = e // _NSUB
    nfull = ept // k
    erem = ept % k
    npair = nfull // 2
    odd_full = nfull % 2
    rpt = n // _NSUB
    nz = rpt // k
    zrem = rpt % k
    mesh = plsc.VectorSubcoreMesh(core_axis_name="c", subcore_axis_name="s")

    @functools.partial(
        pl.kernel,
        out_type=[jax.ShapeDtypeStruct((n, w), _F32),
                  jax.ShapeDtypeStruct((n, w), _F32)],
        mesh=mesh,
        scratch_types=[
            pltpu.VMEM((ept,), jnp.int32),
            pltpu.VMEM((ept,), jnp.int32),
            pltpu.VMEM((k, w), _F32),
            pltpu.VMEM((k, w), _F32),
            pltpu.VMEM_SHARED((n, w), _F32),
            pltpu.SemaphoreType.DMA,
            pltpu.SemaphoreType.DMA,
        ],
        compiler_params=pltpu.CompilerParams(use_tc_tiling_on_sc=False),
    )
    def sc_segsum(ya, yb, ei, outa, outb, srcI, dstI, rowsA, rowsB,
                  acc, semA, semB):
        cid = lax.axis_index("c")
        sid = lax.axis_index("s")

        def start_gather(idx, rows, sem):
            @pl.when(cid == 0)
            def _():
                pltpu.async_copy(ya.at[idx], rows, sem)

            @pl.when(cid == 1)
            def _():
                pltpu.async_copy(yb.at[idx], rows, sem)

        def wait_gather(idx, rows, sem):
            @pl.when(cid == 0)
            def _():
                pltpu.make_async_copy(ya.at[idx], rows, sem).wait()

            @pl.when(cid == 1)
            def _():
                pltpu.make_async_copy(yb.at[idx], rows, sem).wait()

        base_e = sid * ept
        pltpu.sync_copy(ei.at[0, pl.ds(base_e, ept)], srcI)
        pltpu.sync_copy(ei.at[1, pl.ds(base_e, ept)], dstI)

        zero = jnp.zeros((16,), _F32)

        def zrow(i, carry):
            def zcol(j, c2):
                rowsA[i, pl.ds(j * 16, 16)] = zero
                return c2
            return lax.fori_loop(0, w // 16, zcol, carry)

        lax.fori_loop(0, k, zrow, 0)
        base_r = sid * rpt
        for j in range(nz):
            pltpu.sync_copy(rowsA, acc.at[pl.ds(base_r + j * k, k)])
        if zrem:
            pltpu.sync_copy(rowsA.at[pl.ds(0, zrem)],
                            acc.at[pl.ds(base_r + nz * k, zrem)])
        plsc.subcore_barrier()

        if nfull >= 1:
            start_gather(srcI.at[pl.ds(0, k)], rowsA, semA)

        def pair(t, carry):
            off0 = 2 * t * k
            start_gather(srcI.at[pl.ds(off0 + k, k)], rowsB, semB)
            wait_gather(srcI.at[pl.ds(off0, k)], rowsA, semA)
            pltpu.sync_copy(rowsA, acc.at[dstI.at[pl.ds(off0, k)]],
                            add=True)

            @pl.when(2 * t + 2 < nfull)
            def _():
                start_gather(srcI.at[pl.ds(off0 + 2 * k, k)], rowsA, semA)

            wait_gather(srcI.at[pl.ds(off0 + k, k)], rowsB, semB)
            pltpu.sync_copy(rowsB, acc.at[dstI.at[pl.ds(off0 + k, k)]],
                            add=True)
            return carry

        if npair:
            lax.fori_loop(0, npair, pair, 0)
        if odd_full:
            offl = (nfull - 1) * k
            wait_gather(srcI.at[pl.ds(offl, k)], rowsA, semA)
            pltpu.sync_copy(rowsA, acc.at[dstI.at[pl.ds(offl, k)]],
                            add=True)
        if erem:
            offr = nfull * k
            rslice = rowsB.at[pl.ds(0, erem)]
            start_gather(srcI.at[pl.ds(offr, erem)], rslice, semB)
            wait_gather(srcI.at[pl.ds(offr, erem)], rslice, semB)
            pltpu.sync_copy(rslice, acc.at[dstI.at[pl.ds(offr, erem)]],
                            add=True)
        plsc.subcore_barrier()

        @pl.when(cid == 0)
        def _():
            pltpu.sync_copy(acc.at[pl.ds(base_r, rpt)],
                            outa.at[pl.ds(base_r, rpt)])

        @pl.when(cid == 1)
        def _():
            pltpu.sync_copy(acc.at[pl.ds(base_r, rpt)],
                            outb.at[pl.ds(base_r, rpt)])

    return sc_segsum


def _make_sc_count(n, e, kc):
    half = e // 2
    ept = half // _NSUB
    nchunks = ept // kc
    rpt = n // _NSUB
    nz = rpt // kc
    zrem = rpt % kc
    mesh = plsc.VectorSubcoreMesh(core_axis_name="c", subcore_axis_name="s")

    @functools.partial(
        pl.kernel,
        out_type=[jax.ShapeDtypeStruct((n, 16), _F32),
                  jax.ShapeDtypeStruct((n, 16), _F32)],
        mesh=mesh,
        scratch_types=[
            pltpu.VMEM((kc,), jnp.int32),
            pltpu.VMEM((kc, 16), _F32),
            pltpu.VMEM_SHARED((n, 16), _F32),
        ],
        compiler_params=pltpu.CompilerParams(use_tc_tiling_on_sc=False),
    )
    def sc_count(ei, cnta, cntb, dstb, ones, acc):
        cid = lax.axis_index("c")
        sid = lax.axis_index("s")

        zero = jnp.zeros((16,), _F32)

        def zrow(i, carry):
            ones[i, pl.ds(0, 16)] = zero
            return carry

        lax.fori_loop(0, kc, zrow, 0)
        base_r = sid * rpt
        for j in range(nz):
            pltpu.sync_copy(ones, acc.at[pl.ds(base_r + j * kc, kc)])
        if zrem:
            pltpu.sync_copy(ones.at[pl.ds(0, zrem)],
                            acc.at[pl.ds(base_r + nz * kc, zrem)])

        one = jnp.ones((16,), _F32)

        def orow(i, carry):
            ones[i, pl.ds(0, 16)] = one
            return carry

        lax.fori_loop(0, kc, orow, 0)
        plsc.subcore_barrier()

        base_e = cid * half + sid * ept

        def chunk(i, carry):
            pltpu.sync_copy(ei.at[1, pl.ds(base_e + i * kc, kc)], dstb)
            pltpu.sync_copy(ones, acc.at[dstb], add=True)
            return carry

        lax.fori_loop(0, nchunks, chunk, 0)
        plsc.subcore_barrier()

        @pl.when(cid == 0)
        def _():
            pltpu.sync_copy(acc.at[pl.ds(base_r, rpt)],
                            cnta.at[pl.ds(base_r, rpt)])

        @pl.when(cid == 1)
        def _():
            pltpu.sync_copy(acc.at[pl.ds(base_r, rpt)],
                            cntb.at[pl.ds(base_r, rpt)])

    return sc_count


def _tc1_body(x_ref, wl_ref, ya_ref, yb_ref):
    t = jnp.dot(x_ref[...], wl_ref[...], preferred_element_type=_F32)
    ya_ref[...] = t[:, :128]
    yb_ref[...] = t[:, 128:]


def _tc1r_body(x_ref, wr_ref, xr_ref):
    xr_ref[...] = jnp.dot(x_ref[...], wr_ref[...],
                          preferred_element_type=_F32)


def _tc2_body(sa_ref, sb_ref, ca_ref, cb_ref, xr_ref, b1_ref, w2l_ref,
              w2r_ref, y2a_ref, y2b_ref, hrc_ref):
    sa = sa_ref[...]
    cnt = jnp.maximum(ca_ref[:, :1] + cb_ref[:, :1], 1.0)
    h = (jnp.concatenate([sa, sb_ref[...]], axis=1) / cnt
         + xr_ref[...] + b1_ref[...])
    h = jnp.maximum(h, 0.0)
    y2 = jnp.dot(h, w2l_ref[...], preferred_element_type=_F32)
    hr = jnp.dot(h, w2r_ref[...], preferred_element_type=_F32)
    y2a_ref[...] = y2[:, :64]
    y2b_ref[...] = y2[:, 64:]
    zeros = jnp.zeros((sa.shape[0], 15), _F32)
    hrc_ref[...] = jnp.concatenate([hr, cnt, zeros], axis=1)


def _tc3_body(s2a_ref, s2b_ref, hrc_ref, b2_ref, out_ref):
    hrc = hrc_ref[...]
    cnt = hrc[:, 128:129]
    pre = (jnp.concatenate([s2a_ref[...], s2b_ref[...]], axis=1) / cnt
           + hrc[:, :128] + b2_ref[...])
    mx = jnp.max(pre, axis=1, keepdims=True)
    lse = jnp.log(jnp.sum(jnp.exp(pre - mx), axis=1, keepdims=True))
    out_ref[...] = pre - mx - lse


def kernel(x, edge_index, W1_l, W1_r, b1, W2_l, W2_r, b2):
    n, d_in = x.shape
    e = edge_index.shape[1]
    d_h = W1_l.shape[1]
    d_out = W2_l.shape[1]
    rb = 1000
    grid = (n // rb,)
    row_spec = lambda w_: pl.BlockSpec((rb, w_), lambda r: (r, 0))
    full_spec = lambda a, b_: pl.BlockSpec((a, b_), lambda r: (0, 0))

    cnta, cntb = _make_sc_count(n, e, 1000)(edge_index)

    ya, yb = pl.pallas_call(
        _tc1_body,
        grid=grid,
        in_specs=[row_spec(d_in), full_spec(d_in, d_h)],
        out_specs=[row_spec(128), row_spec(128)],
        out_shape=[jax.ShapeDtypeStruct((n, 128), _F32),
                   jax.ShapeDtypeStruct((n, 128), _F32)],
    )(x, W1_l)

    sa, sb = _make_sc_segsum(n, e, 128, 120)(ya, yb, edge_index)

    xr = pl.pallas_call(
        _tc1r_body,
        grid=grid,
        in_specs=[row_spec(d_in), full_spec(d_in, d_h)],
        out_specs=row_spec(d_h),
        out_shape=jax.ShapeDtypeStruct((n, d_h), _F32),
    )(x, W1_r)

    y2a, y2b, hrc = pl.pallas_call(
        _tc2_body,
        grid=grid,
        in_specs=[row_spec(128), row_spec(128), row_spec(16), row_spec(16),
                  row_spec(d_h), full_spec(1, d_h), full_spec(d_h, d_out),
                  full_spec(d_h, d_out)],
        out_specs=[row_spec(64), row_spec(64), row_spec(144)],
        out_shape=[jax.ShapeDtypeStruct((n, 64), _F32),
                   jax.ShapeDtypeStruct((n, 64), _F32),
                   jax.ShapeDtypeStruct((n, 144), _F32)],
    )(sa, sb, cnta, cntb, xr, b1.reshape(1, d_h), W2_l, W2_r)

    s2a, s2b = _make_sc_segsum(n, e, 64, 552)(y2a, y2b, edge_index)

    out = pl.pallas_call(
        _tc3_body,
        grid=grid,
        in_specs=[row_spec(64), row_spec(64), row_spec(144),
                  full_spec(1, d_out)],
        out_specs=row_spec(d_out),
        out_shape=jax.ShapeDtypeStruct((n, d_out), _F32),
    )(s2a, s2b, hrc, b2.reshape(1, d_out))
    return out


# speedup vs baseline: 1.0654x; 1.0654x over previous
import functools

import jax
import jax.numpy as jnp
from jax import lax
from jax.experimental import pallas as pl
from jax.experimental.pallas import tpu as pltpu
from jax.experimental.pallas import tpu_sc as plsc

_NSUB = 16
_F32 = jnp.float32


def _make_sc_segsum(n, e, w, k):
    ept = e // _NSUB
    nfull = ept // k
    erem = ept % k
    npair = nfull // 2
    odd_full = nfull % 2
    rpt = n // _NSUB
    nz = rpt // k
    zrem = rpt % k
    mesh = plsc.VectorSubcoreMesh(core_axis_name="c", subcore_axis_name="s")

    @functools.partial(
        pl.kernel,
        out_type=[jax.ShapeDtypeStruct((n, w), _F32),
                  jax.ShapeDtypeStruct((n, w), _F32)],
        mesh=mesh,
        scratch_types=[
            pltpu.VMEM((ept,), jnp.int32),
            pltpu.VMEM((ept,), jnp.int32),
            pltpu.VMEM((k, w), _F32),
            pltpu.VMEM((k, w), _F32),
            pltpu.VMEM_SHARED((n, w), _F32),
            pltpu.SemaphoreType.DMA,
            pltpu.SemaphoreType.DMA,
        ],
        compiler_params=pltpu.CompilerParams(use_tc_tiling_on_sc=False),
    )
    def sc_segsum(ya, yb, ei, outa, outb, srcI, dstI, rowsA, rowsB,
                  acc, semA, semB):
        cid = lax.axis_index("c")
        sid = lax.axis_index("s")

        def start_gather(idx, rows, sem):
            @pl.when(cid == 0)
            def _():
                pltpu.async_copy(ya.at[idx], rows, sem)

            @pl.when(cid == 1)
            def _():
                pltpu.async_copy(yb.at[idx], rows, sem)

        def wait_gather(idx, rows, sem):
            @pl.when(cid == 0)
            def _():
                pltpu.make_async_copy(ya.at[idx], rows, sem).wait()

            @pl.when(cid == 1)
            def _():
                pltpu.make_async_copy(yb.at[idx], rows, sem).wait()

        base_e = sid * ept
        pltpu.sync_copy(ei.at[0, pl.ds(base_e, ept)], srcI)
        pltpu.sync_copy(ei.at[1, pl.ds(base_e, ept)], dstI)

        zero = jnp.zeros((16,), _F32)

        def zrow(i, carry):
            def zcol(j, c2):
                rowsA[i, pl.ds(j * 16, 16)] = zero
                return c2
            return lax.fori_loop(0, w // 16, zcol, carry)

        lax.fori_loop(0, k, zrow, 0)
        base_r = sid * rpt
        for j in range(nz):
            pltpu.sync_copy(rowsA, acc.at[pl.ds(base_r + j * k, k)])
        if zrem:
            pltpu.sync_copy(rowsA.at[pl.ds(0, zrem)],
                            acc.at[pl.ds(base_r + nz * k, zrem)])
        plsc.subcore_barrier()

        if nfull >= 1:
            start_gather(srcI.at[pl.ds(0, k)], rowsA, semA)

        def pair(t, carry):
            off0 = 2 * t * k
            start_gather(srcI.at[pl.ds(off0 + k, k)], rowsB, semB)
            wait_gather(srcI.at[pl.ds(off0, k)], rowsA, semA)
            pltpu.sync_copy(rowsA, acc.at[dstI.at[pl.ds(off0, k)]],
                            add=True)

            @pl.when(2 * t + 2 < nfull)
            def _():
                start_gather(srcI.at[pl.ds(off0 + 2 * k, k)], rowsA, semA)

            wait_gather(srcI.at[pl.ds(off0 + k, k)], rowsB, semB)
            pltpu.sync_copy(rowsB, acc.at[dstI.at[pl.ds(off0 + k, k)]],
                            add=True)
            return carry

        if npair:
            lax.fori_loop(0, npair, pair, 0)
        if odd_full:
            offl = (nfull - 1) * k
            wait_gather(srcI.at[pl.ds(offl, k)], rowsA, semA)
            pltpu.sync_copy(rowsA, acc.at[dstI.at[pl.ds(offl, k)]],
                            add=True)
        if erem:
            offr = nfull * k
            rslice = rowsB.at[pl.ds(0, erem)]
            start_gather(srcI.at[pl.ds(offr, erem)], rslice, semB)
            wait_gather(srcI.at[pl.ds(offr, erem)], rslice, semB)
            pltpu.sync_copy(rslice, acc.at[dstI.at[pl.ds(offr, erem)]],
                            add=True)
        plsc.subcore_barrier()

        @pl.when(cid == 0)
        def _():
            pltpu.sync_copy(acc.at[pl.ds(base_r, rpt)],
                            outa.at[pl.ds(base_r, rpt)])

        @pl.when(cid == 1)
        def _():
            pltpu.sync_copy(acc.at[pl.ds(base_r, rpt)],
                            outb.at[pl.ds(base_r, rpt)])

    return sc_segsum


def _make_sc_segsum_es(n, e, w, k):
    half = e // 2
    ept = half // _NSUB
    nfull = ept // k
    erem = ept % k
    npair = nfull // 2
    odd_full = nfull % 2
    rpt = n // _NSUB
    nz = rpt // k
    zrem = rpt % k
    mesh = plsc.VectorSubcoreMesh(core_axis_name="c", subcore_axis_name="s")

    @functools.partial(
        pl.kernel,
        out_type=[jax.ShapeDtypeStruct((n, w), _F32),
                  jax.ShapeDtypeStruct((n, w), _F32)],
        mesh=mesh,
        scratch_types=[
            pltpu.VMEM((ept,), jnp.int32),
            pltpu.VMEM((ept,), jnp.int32),
            pltpu.VMEM((k, w), _F32),
            pltpu.VMEM((k, w), _F32),
            pltpu.VMEM_SHARED((n, w), _F32),
            pltpu.SemaphoreType.DMA,
            pltpu.SemaphoreType.DMA,
        ],
        compiler_params=pltpu.CompilerParams(use_tc_tiling_on_sc=False),
    )
    def sc_segsum_es(y, ei, outa, outb, srcI, dstI, rowsA, rowsB,
                     acc, semA, semB):
        cid = lax.axis_index("c")
        sid = lax.axis_index("s")

        base_e = cid * half + sid * ept
        pltpu.sync_copy(ei.at[0, pl.ds(base_e, ept)], srcI)
        pltpu.sync_copy(ei.at[1, pl.ds(base_e, ept)], dstI)

        zero = jnp.zeros((16,), _F32)

        def zrow(i, carry):
            def zcol(j, c2):
                rowsA[i, pl.ds(j * 16, 16)] = zero
                return c2
            return lax.fori_loop(0, w // 16, zcol, carry)

        lax.fori_loop(0, k, zrow, 0)
        base_r = sid * rpt
        for j in range(nz):
            pltpu.sync_copy(rowsA, acc.at[pl.ds(base_r + j * k, k)])
        if zrem:
            pltpu.sync_copy(rowsA.at[pl.ds(0, zrem)],
                            acc.at[pl.ds(base_r + nz * k, zrem)])
        plsc.subcore_barrier()

        if nfull >= 1:
            pltpu.async_copy(y.at[srcI.at[pl.ds(0, k)]], rowsA, semA)

        def pair(t, carry):
            off0 = 2 * t * k
            pltpu.async_copy(y.at[srcI.at[pl.ds(off0 + k, k)]], rowsB,
                             semB)
            pltpu.make_async_copy(y.at[srcI.at[pl.ds(off0, k)]], rowsA,
                                  semA).wait()
            pltpu.sync_copy(rowsA, acc.at[dstI.at[pl.ds(off0, k)]],
                            add=True)

            @pl.when(2 * t + 2 < nfull)
            def _():
                pltpu.async_copy(y.at[srcI.at[pl.ds(off0 + 2 * k, k)]],
                                 rowsA, semA)

            pltpu.make_async_copy(y.at[srcI.at[pl.ds(off0 + k, k)]],
                                  rowsB, semB).wait()
            pltpu.sync_copy(rowsB, acc.at[dstI.at[pl.ds(off0 + k, k)]],
                            add=True)
            return carry

        if npair:
            lax.fori_loop(0, npair, pair, 0)
        if odd_full:
            offl = (nfull - 1) * k
            pltpu.make_async_copy(y.at[srcI.at[pl.ds(offl, k)]], rowsA,
                                  semA).wait()
            pltpu.sync_copy(rowsA, acc.at[dstI.at[pl.ds(offl, k)]],
                            add=True)
        if erem:
            offr = nfull * k
            rslice = rowsB.at[pl.ds(0, erem)]
            pltpu.async_copy(y.at[srcI.at[pl.ds(offr, erem)]], rslice,
                             semB)
            pltpu.make_async_copy(y.at[srcI.at[pl.ds(offr, erem)]],
                                  rslice, semB).wait()
            pltpu.sync_copy(rslice, acc.at[dstI.at[pl.ds(offr, erem)]],
                            add=True)
        plsc.subcore_barrier()

        @pl.when(cid == 0)
        def _():
            pltpu.sync_copy(acc.at[pl.ds(base_r, rpt)],
                            outa.at[pl.ds(base_r, rpt)])

        @pl.when(cid == 1)
        def _():
            pltpu.sync_copy(acc.at[pl.ds(base_r, rpt)],
                            outb.at[pl.ds(base_r, rpt)])

    return sc_segsum_es


def _make_sc_count(n, e, kc):
    half = e // 2
    ept = half // _NSUB
    nchunks = ept // kc
    rpt = n // _NSUB
    nz = rpt // kc
    zrem = rpt % kc
    mesh = plsc.VectorSubcoreMesh(core_axis_name="c", subcore_axis_name="s")

    @functools.partial(
        pl.kernel,
        out_type=[jax.ShapeDtypeStruct((n, 16), _F32),
                  jax.ShapeDtypeStruct((n, 16), _F32)],
        mesh=mesh,
        scratch_types=[
            pltpu.VMEM((kc,), jnp.int32),
            pltpu.VMEM((kc, 16), _F32),
            pltpu.VMEM_SHARED((n, 16), _F32),
        ],
        compiler_params=pltpu.CompilerParams(use_tc_tiling_on_sc=False),
    )
    def sc_count(ei, cnta, cntb, dstb, ones, acc):
        cid = lax.axis_index("c")
        sid = lax.axis_index("s")

        zero = jnp.zeros((16,), _F32)

        def zrow(i, carry):
            ones[i, pl.ds(0, 16)] = zero
            return carry

        lax.fori_loop(0, kc, zrow, 0)
        base_r = sid * rpt
        for j in range(nz):
            pltpu.sync_copy(ones, acc.at[pl.ds(base_r + j * kc, kc)])
        if zrem:
            pltpu.sync_copy(ones.at[pl.ds(0, zrem)],
                            acc.at[pl.ds(base_r + nz * kc, zrem)])

        one = jnp.ones((16,), _F32)

        def orow(i, carry):
            ones[i, pl.ds(0, 16)] = one
            return carry

        lax.fori_loop(0, kc, orow, 0)
        plsc.subcore_barrier()

        base_e = cid * half + sid * ept

        def chunk(i, carry):
            pltpu.sync_copy(ei.at[1, pl.ds(base_e + i * kc, kc)], dstb)
            pltpu.sync_copy(ones, acc.at[dstb], add=True)
            return carry

        lax.fori_loop(0, nchunks, chunk, 0)
        plsc.subcore_barrier()

        @pl.when(cid == 0)
        def _():
            pltpu.sync_copy(acc.at[pl.ds(base_r, rpt)],
                            cnta.at[pl.ds(base_r, rpt)])

        @pl.when(cid == 1)
        def _():
            pltpu.sync_copy(acc.at[pl.ds(base_r, rpt)],
                            cntb.at[pl.ds(base_r, rpt)])

    return sc_count


def _tc1_body(x_ref, wl_ref, wr_ref, ya_ref, yb_ref, xr_ref):
    xb = x_ref[...]
    t = jnp.dot(xb, wl_ref[...], preferred_element_type=_F32)
    ya_ref[...] = t[:, :128]
    yb_ref[...] = t[:, 128:]
    xr_ref[...] = jnp.dot(xb, wr_ref[...], preferred_element_type=_F32)


def _tc2_body(sa_ref, sb_ref, ca_ref, cb_ref, xr_ref, b1_ref, w2l_ref,
              w2r_ref, y2_ref, hrc_ref):
    sa = sa_ref[...]
    cnt = jnp.maximum(ca_ref[:, :1] + cb_ref[:, :1], 1.0)
    h = (jnp.concatenate([sa, sb_ref[...]], axis=1) / cnt
         + xr_ref[...] + b1_ref[...])
    h = jnp.maximum(h, 0.0)
    y2_ref[...] = jnp.dot(h, w2l_ref[...], preferred_element_type=_F32)
    hr = jnp.dot(h, w2r_ref[...], preferred_element_type=_F32)
    zeros = jnp.zeros((sa.shape[0], 15), _F32)
    hrc_ref[...] = jnp.concatenate([hr, cnt, zeros], axis=1)


def _tc3_body(s2a_ref, s2b_ref, hrc_ref, b2_ref, out_ref):
    hrc = hrc_ref[...]
    cnt = hrc[:, 128:129]
    pre = ((s2a_ref[...] + s2b_ref[...]) / cnt
           + hrc[:, :128] + b2_ref[...])
    mx = jnp.max(pre, axis=1, keepdims=True)
    lse = jnp.log(jnp.sum(jnp.exp(pre - mx), axis=1, keepdims=True))
    out_ref[...] = pre - mx - lse


def kernel(x, edge_index, W1_l, W1_r, b1, W2_l, W2_r, b2):
    n, d_in = x.shape
    e = edge_index.shape[1]
    d_h = W1_l.shape[1]
    d_out = W2_l.shape[1]
    rb = 1000
    grid = (n // rb,)
    row_spec = lambda w_: pl.BlockSpec((rb, w_), lambda r: (r, 0))
    full_spec = lambda a, b_: pl.BlockSpec((a, b_), lambda r: (0, 0))

    cnta, cntb = _make_sc_count(n, e, 1000)(edge_index)

    ya, yb, xr = pl.pallas_call(
        _tc1_body,
        grid=grid,
        in_specs=[row_spec(d_in), full_spec(d_in, d_h), full_spec(d_in, d_h)],
        out_specs=[row_spec(128), row_spec(128), row_spec(d_h)],
        out_shape=[jax.ShapeDtypeStruct((n, 128), _F32),
                   jax.ShapeDtypeStruct((n, 128), _F32),
                   jax.ShapeDtypeStruct((n, d_h), _F32)],
    )(x, W1_l, W1_r)

    sa, sb = _make_sc_segsum(n, e, 128, 120)(ya, yb, edge_index)

    y2, hrc = pl.pallas_call(
        _tc2_body,
        grid=grid,
        in_specs=[row_spec(128), row_spec(128), row_spec(16), row_spec(16),
                  row_spec(d_h), full_spec(1, d_h), full_spec(d_h, d_out),
                  full_spec(d_h, d_out)],
        out_specs=[row_spec(128), row_spec(144)],
        out_shape=[jax.ShapeDtypeStruct((n, 128), _F32),
                   jax.ShapeDtypeStruct((n, 144), _F32)],
    )(sa, sb, cnta, cntb, xr, b1.reshape(1, d_h), W2_l, W2_r)

    s2a, s2b = _make_sc_segsum_es(n, e, 128, 128)(y2, edge_index)

    out = pl.pallas_call(
        _tc3_body,
        grid=grid,
        in_specs=[row_spec(128), row_spec(128), row_spec(144),
                  full_spec(1, d_out)],
        out_specs=row_spec(d_out),
        out_shape=jax.ShapeDtypeStruct((n, d_out), _F32),
    )(s2a, s2b, hrc, b2.reshape(1, d_out))
    return out


# speedup vs baseline: 1.0822x; 1.0157x over previous
import functools

import jax
import jax.numpy as jnp
from jax import lax
from jax.experimental import pallas as pl
from jax.experimental.pallas import tpu as pltpu
from jax.experimental.pallas import tpu_sc as plsc

_NSUB = 16
_F32 = jnp.float32


def _make_sc_segsum(n, e, w, k):
    ept = e // _NSUB
    nfull = ept // k
    erem = ept % k
    npair = nfull // 2
    odd_full = nfull % 2
    rpt = n // _NSUB
    nz = rpt // k
    zrem = rpt % k
    mesh = plsc.VectorSubcoreMesh(core_axis_name="c", subcore_axis_name="s")

    @functools.partial(
        pl.kernel,
        out_type=[jax.ShapeDtypeStruct((n, w), _F32),
                  jax.ShapeDtypeStruct((n, w), _F32)],
        mesh=mesh,
        scratch_types=[
            pltpu.VMEM((ept,), jnp.int32),
            pltpu.VMEM((ept,), jnp.int32),
            pltpu.VMEM((k, w), _F32),
            pltpu.VMEM((k, w), _F32),
            pltpu.VMEM_SHARED((n, w), _F32),
            pltpu.SemaphoreType.DMA,
            pltpu.SemaphoreType.DMA,
        ],
        compiler_params=pltpu.CompilerParams(use_tc_tiling_on_sc=False),
    )
    def sc_segsum(ya, yb, ei, outa, outb, srcI, dstI, rowsA, rowsB,
                  acc, semA, semB):
        cid = lax.axis_index("c")
        sid = lax.axis_index("s")

        def start_gather(idx, rows, sem):
            @pl.when(cid == 0)
            def _():
                pltpu.async_copy(ya.at[idx], rows, sem)

            @pl.when(cid == 1)
            def _():
                pltpu.async_copy(yb.at[idx], rows, sem)

        def wait_gather(idx, rows, sem):
            @pl.when(cid == 0)
            def _():
                pltpu.make_async_copy(ya.at[idx], rows, sem).wait()

            @pl.when(cid == 1)
            def _():
                pltpu.make_async_copy(yb.at[idx], rows, sem).wait()

        base_e = sid * ept
        pltpu.sync_copy(ei.at[0, pl.ds(base_e, ept)], srcI)
        pltpu.sync_copy(ei.at[1, pl.ds(base_e, ept)], dstI)

        zero = jnp.zeros((16,), _F32)

        def zrow(i, carry):
            def zcol(j, c2):
                rowsA[i, pl.ds(j * 16, 16)] = zero
                return c2
            return lax.fori_loop(0, w // 16, zcol, carry)

        lax.fori_loop(0, k, zrow, 0)
        base_r = sid * rpt
        for j in range(nz):
            pltpu.sync_copy(rowsA, acc.at[pl.ds(base_r + j * k, k)])
        if zrem:
            pltpu.sync_copy(rowsA.at[pl.ds(0, zrem)],
                            acc.at[pl.ds(base_r + nz * k, zrem)])
        plsc.subcore_barrier()

        if nfull >= 1:
            start_gather(srcI.at[pl.ds(0, k)], rowsA, semA)

        def pair(t, carry):
            off0 = 2 * t * k
            start_gather(srcI.at[pl.ds(off0 + k, k)], rowsB, semB)
            wait_gather(srcI.at[pl.ds(off0, k)], rowsA, semA)
            pltpu.sync_copy(rowsA, acc.at[dstI.at[pl.ds(off0, k)]],
                            add=True)

            @pl.when(2 * t + 2 < nfull)
            def _():
                start_gather(srcI.at[pl.ds(off0 + 2 * k, k)], rowsA, semA)

            wait_gather(srcI.at[pl.ds(off0 + k, k)], rowsB, semB)
            pltpu.sync_copy(rowsB, acc.at[dstI.at[pl.ds(off0 + k, k)]],
                            add=True)
            return carry

        if npair:
            lax.fori_loop(0, npair, pair, 0)
        if odd_full:
            offl = (nfull - 1) * k
            wait_gather(srcI.at[pl.ds(offl, k)], rowsA, semA)
            pltpu.sync_copy(rowsA, acc.at[dstI.at[pl.ds(offl, k)]],
                            add=True)
        if erem:
            offr = nfull * k
            rslice = rowsB.at[pl.ds(0, erem)]
            start_gather(srcI.at[pl.ds(offr, erem)], rslice, semB)
            wait_gather(srcI.at[pl.ds(offr, erem)], rslice, semB)
            pltpu.sync_copy(rslice, acc.at[dstI.at[pl.ds(offr, erem)]],
                            add=True)
        plsc.subcore_barrier()

        @pl.when(cid == 0)
        def _():
            pltpu.sync_copy(acc.at[pl.ds(base_r, rpt)],
                            outa.at[pl.ds(base_r, rpt)])

        @pl.when(cid == 1)
        def _():
            pltpu.sync_copy(acc.at[pl.ds(base_r, rpt)],
                            outb.at[pl.ds(base_r, rpt)])

    return sc_segsum


def _make_sc_segsum_es(n, e, w, k):
    half = e // 2
    ept = half // _NSUB
    nfull = ept // k
    erem = ept % k
    npair = nfull // 2
    odd_full = nfull % 2
    rpt = n // _NSUB
    nz = rpt // k
    zrem = rpt % k
    mesh = plsc.VectorSubcoreMesh(core_axis_name="c", subcore_axis_name="s")

    @functools.partial(
        pl.kernel,
        out_type=[jax.ShapeDtypeStruct((n, w), _F32),
                  jax.ShapeDtypeStruct((n, w), _F32)],
        mesh=mesh,
        scratch_types=[
            pltpu.VMEM((ept,), jnp.int32),
            pltpu.VMEM((ept,), jnp.int32),
            pltpu.VMEM((k, w), _F32),
            pltpu.VMEM((k, w), _F32),
            pltpu.VMEM_SHARED((n, w), _F32),
            pltpu.SemaphoreType.DMA,
            pltpu.SemaphoreType.DMA,
        ],
        compiler_params=pltpu.CompilerParams(use_tc_tiling_on_sc=False),
    )
    def sc_segsum_es(y, ei, outa, outb, srcI, dstI, rowsA, rowsB,
                     acc, semA, semB):
        cid = lax.axis_index("c")
        sid = lax.axis_index("s")

        base_e = cid * half + sid * ept
        pltpu.sync_copy(ei.at[0, pl.ds(base_e, ept)], srcI)
        pltpu.sync_copy(ei.at[1, pl.ds(base_e, ept)], dstI)

        zero = jnp.zeros((16,), _F32)

        def zrow(i, carry):
            def zcol(j, c2):
                rowsA[i, pl.ds(j * 16, 16)] = zero
                return c2
            return lax.fori_loop(0, w // 16, zcol, carry)

        lax.fori_loop(0, k, zrow, 0)
        base_r = sid * rpt
        for j in range(nz):
            pltpu.sync_copy(rowsA, acc.at[pl.ds(base_r + j * k, k)])
        if zrem:
            pltpu.sync_copy(rowsA.at[pl.ds(0, zrem)],
                            acc.at[pl.ds(base_r + nz * k, zrem)])
        plsc.subcore_barrier()

        if nfull >= 1:
            pltpu.async_copy(y.at[srcI.at[pl.ds(0, k)]], rowsA, semA)

        def pair(t, carry):
            off0 = 2 * t * k
            pltpu.async_copy(y.at[srcI.at[pl.ds(off0 + k, k)]], rowsB,
                             semB)
            pltpu.make_async_copy(y.at[srcI.at[pl.ds(off0, k)]], rowsA,
                                  semA).wait()
            pltpu.sync_copy(rowsA, acc.at[dstI.at[pl.ds(off0, k)]],
                            add=True)

            @pl.when(2 * t + 2 < nfull)
            def _():
                pltpu.async_copy(y.at[srcI.at[pl.ds(off0 + 2 * k, k)]],
                                 rowsA, semA)

            pltpu.make_async_copy(y.at[srcI.at[pl.ds(off0 + k, k)]],
                                  rowsB, semB).wait()
            pltpu.sync_copy(rowsB, acc.at[dstI.at[pl.ds(off0 + k, k)]],
                            add=True)
            return carry

        if npair:
            lax.fori_loop(0, npair, pair, 0)
        if odd_full:
            offl = (nfull - 1) * k
            pltpu.make_async_copy(y.at[srcI.at[pl.ds(offl, k)]], rowsA,
                                  semA).wait()
            pltpu.sync_copy(rowsA, acc.at[dstI.at[pl.ds(offl, k)]],
                            add=True)
        if erem:
            offr = nfull * k
            rslice = rowsB.at[pl.ds(0, erem)]
            pltpu.async_copy(y.at[srcI.at[pl.ds(offr, erem)]], rslice,
                             semB)
            pltpu.make_async_copy(y.at[srcI.at[pl.ds(offr, erem)]],
                                  rslice, semB).wait()
            pltpu.sync_copy(rslice, acc.at[dstI.at[pl.ds(offr, erem)]],
                            add=True)
        plsc.subcore_barrier()

        @pl.when(cid == 0)
        def _():
            pltpu.sync_copy(acc.at[pl.ds(base_r, rpt)],
                            outa.at[pl.ds(base_r, rpt)])

        @pl.when(cid == 1)
        def _():
            pltpu.sync_copy(acc.at[pl.ds(base_r, rpt)],
                            outb.at[pl.ds(base_r, rpt)])

    return sc_segsum_es


def _make_sc_count(n, e, kc):
    half = e // 2
    ept = half // _NSUB
    nchunks = ept // kc
    rpt = n // _NSUB
    nz = rpt // kc
    zrem = rpt % kc
    mesh = plsc.VectorSubcoreMesh(core_axis_name="c", subcore_axis_name="s")

    @functools.partial(
        pl.kernel,
        out_type=[jax.ShapeDtypeStruct((n, 16), _F32),
                  jax.ShapeDtypeStruct((n, 16), _F32)],
        mesh=mesh,
        scratch_types=[
            pltpu.VMEM((kc,), jnp.int32),
            pltpu.VMEM((kc, 16), _F32),
            pltpu.VMEM_SHARED((n, 16), _F32),
        ],
        compiler_params=pltpu.CompilerParams(use_tc_tiling_on_sc=False),
    )
    def sc_count(ei, cnta, cntb, dstb, ones, acc):
        cid = lax.axis_index("c")
        sid = lax.axis_index("s")

        zero = jnp.zeros((16,), _F32)

        def zrow(i, carry):
            ones[i, pl.ds(0, 16)] = zero
            return carry

        lax.fori_loop(0, kc, zrow, 0)
        base_r = sid * rpt
        for j in range(nz):
            pltpu.sync_copy(ones, acc.at[pl.ds(base_r + j * kc, kc)])
        if zrem:
            pltpu.sync_copy(ones.at[pl.ds(0, zrem)],
                            acc.at[pl.ds(base_r + nz * kc, zrem)])

        one = jnp.ones((16,), _F32)

        def orow(i, carry):
            ones[i, pl.ds(0, 16)] = one
            return carry

        lax.fori_loop(0, kc, orow, 0)
        plsc.subcore_barrier()

        base_e = cid * half + sid * ept

        def chunk(i, carry):
            pltpu.sync_copy(ei.at[1, pl.ds(base_e + i * kc, kc)], dstb)
            pltpu.sync_copy(ones, acc.at[dstb], add=True)
            return carry

        lax.fori_loop(0, nchunks, chunk, 0)
        plsc.subcore_barrier()

        @pl.when(cid == 0)
        def _():
            pltpu.sync_copy(acc.at[pl.ds(base_r, rpt)],
                            cnta.at[pl.ds(base_r, rpt)])

        @pl.when(cid == 1)
        def _():
            pltpu.sync_copy(acc.at[pl.ds(base_r, rpt)],
                            cntb.at[pl.ds(base_r, rpt)])

    return sc_count


def _tc1_body(x_ref, wl_ref, ya_ref, yb_ref):
    t = jnp.dot(x_ref[...], wl_ref[...], preferred_element_type=_F32)
    ya_ref[...] = t[:, :128]
    yb_ref[...] = t[:, 128:]


def _tc2_body(sa_ref, sb_ref, ca_ref, cb_ref, x_ref, w1r_ref, b1_ref,
              w2l_ref, w2r_ref, y2_ref, hrc_ref):
    sa = sa_ref[...]
    cnt = jnp.maximum(ca_ref[:, :1] + cb_ref[:, :1], 1.0)
    xr = jnp.dot(x_ref[...], w1r_ref[...], preferred_element_type=_F32)
    h = (jnp.concatenate([sa, sb_ref[...]], axis=1) / cnt
         + xr + b1_ref[...])
    h = jnp.maximum(h, 0.0)
    y2_ref[...] = jnp.dot(h, w2l_ref[...], preferred_element_type=_F32)
    hr = jnp.dot(h, w2r_ref[...], preferred_element_type=_F32)
    zeros = jnp.zeros((sa.shape[0], 15), _F32)
    hrc_ref[...] = jnp.concatenate([hr, cnt, zeros], axis=1)


def _tc3_body(s2a_ref, s2b_ref, hrc_ref, b2_ref, out_ref):
    hrc = hrc_ref[...]
    cnt = hrc[:, 128:129]
    pre = ((s2a_ref[...] + s2b_ref[...]) / cnt
           + hrc[:, :128] + b2_ref[...])
    mx = jnp.max(pre, axis=1, keepdims=True)
    lse = jnp.log(jnp.sum(jnp.exp(pre - mx), axis=1, keepdims=True))
    out_ref[...] = pre - mx - lse


def kernel(x, edge_index, W1_l, W1_r, b1, W2_l, W2_r, b2):
    n, d_in = x.shape
    e = edge_index.shape[1]
    d_h = W1_l.shape[1]
    d_out = W2_l.shape[1]
    rb = 2000
    grid = (n // rb,)
    row_spec = lambda w_: pl.BlockSpec((rb, w_), lambda r: (r, 0))
    full_spec = lambda a, b_: pl.BlockSpec((a, b_), lambda r: (0, 0))

    cnta, cntb = _make_sc_count(n, e, 1000)(edge_index)

    ya, yb = pl.pallas_call(
        _tc1_body,
        grid=grid,
        in_specs=[row_spec(d_in), full_spec(d_in, d_h)],
        out_specs=[row_spec(128), row_spec(128)],
        out_shape=[jax.ShapeDtypeStruct((n, 128), _F32),
                   jax.ShapeDtypeStruct((n, 128), _F32)],
    )(x, W1_l)

    sa, sb = _make_sc_segsum(n, e, 128, 120)(ya, yb, edge_index)

    y2, hrc = pl.pallas_call(
        _tc2_body,
        grid=grid,
        in_specs=[row_spec(128), row_spec(128), row_spec(16), row_spec(16),
                  row_spec(d_in), full_spec(d_in, d_h), full_spec(1, d_h),
                  full_spec(d_h, d_out), full_spec(d_h, d_out)],
        out_specs=[row_spec(128), row_spec(144)],
        out_shape=[jax.ShapeDtypeStruct((n, 128), _F32),
                   jax.ShapeDtypeStruct((n, 144), _F32)],
    )(sa, sb, cnta, cntb, x, W1_r, b1.reshape(1, d_h), W2_l, W2_r)

    s2a, s2b = _make_sc_segsum_es(n, e, 128, 128)(y2, edge_index)

    out = pl.pallas_call(
        _tc3_body,
        grid=grid,
        in_specs=[row_spec(128), row_spec(128), row_spec(144),
                  full_spec(1, d_out)],
        out_specs=row_spec(d_out),
        out_shape=jax.ShapeDtypeStruct((n, d_out), _F32),
    )(s2a, s2b, hrc, b2.reshape(1, d_out))
    return out


# speedup vs baseline: 1.1018x; 1.0181x over previous
import functools

import jax
import jax.numpy as jnp
from jax import lax
from jax.experimental import pallas as pl
from jax.experimental.pallas import tpu as pltpu
from jax.experimental.pallas import tpu_sc as plsc

_NSUB = 16
_F32 = jnp.float32


def _make_sc_segsum(n, e, w, k):
    ept = e // _NSUB
    nfull = ept // k
    erem = ept % k
    npair = nfull // 2
    odd_full = nfull % 2
    rpt = n // _NSUB
    nz = rpt // k
    zrem = rpt % k
    mesh = plsc.VectorSubcoreMesh(core_axis_name="c", subcore_axis_name="s")

    @functools.partial(
        pl.kernel,
        out_type=[jax.ShapeDtypeStruct((n, w), _F32),
                  jax.ShapeDtypeStruct((n, w), _F32)],
        mesh=mesh,
        scratch_types=[
            pltpu.VMEM((ept,), jnp.int32),
            pltpu.VMEM((ept,), jnp.int32),
            pltpu.VMEM((k, w), _F32),
            pltpu.VMEM((k, w), _F32),
            pltpu.VMEM_SHARED((n, w), _F32),
            pltpu.SemaphoreType.DMA,
            pltpu.SemaphoreType.DMA,
        ],
        compiler_params=pltpu.CompilerParams(use_tc_tiling_on_sc=False),
    )
    def sc_segsum(ya, yb, ei, outa, outb, srcI, dstI, rowsA, rowsB,
                  acc, semA, semB):
        cid = lax.axis_index("c")
        sid = lax.axis_index("s")

        def start_gather(idx, rows, sem):
            @pl.when(cid == 0)
            def _():
                pltpu.async_copy(ya.at[idx], rows, sem)

            @pl.when(cid == 1)
            def _():
                pltpu.async_copy(yb.at[idx], rows, sem)

        def wait_gather(idx, rows, sem):
            @pl.when(cid == 0)
            def _():
                pltpu.make_async_copy(ya.at[idx], rows, sem).wait()

            @pl.when(cid == 1)
            def _():
                pltpu.make_async_copy(yb.at[idx], rows, sem).wait()

        base_e = sid * ept
        pltpu.async_copy(ei.at[0, pl.ds(base_e, ept)], srcI, semA)
        pltpu.async_copy(ei.at[1, pl.ds(base_e, ept)], dstI, semB)

        zero = jnp.zeros((16,), _F32)

        def zrow(i, carry):
            def zcol(j, c2):
                rowsB[i, pl.ds(j * 16, 16)] = zero
                return c2
            return lax.fori_loop(0, w // 16, zcol, carry)

        lax.fori_loop(0, k, zrow, 0)
        base_r = sid * rpt
        for j in range(nz):
            pltpu.sync_copy(rowsB, acc.at[pl.ds(base_r + j * k, k)])
        if zrem:
            pltpu.sync_copy(rowsB.at[pl.ds(0, zrem)],
                            acc.at[pl.ds(base_r + nz * k, zrem)])
        pltpu.make_async_copy(ei.at[0, pl.ds(base_e, ept)], srcI,
                              semA).wait()
        pltpu.make_async_copy(ei.at[1, pl.ds(base_e, ept)], dstI,
                              semB).wait()

        if nfull >= 1:
            start_gather(srcI.at[pl.ds(0, k)], rowsA, semA)
        plsc.subcore_barrier()

        def pair(t, carry):
            off0 = 2 * t * k
            start_gather(srcI.at[pl.ds(off0 + k, k)], rowsB, semB)
            wait_gather(srcI.at[pl.ds(off0, k)], rowsA, semA)
            pltpu.sync_copy(rowsA, acc.at[dstI.at[pl.ds(off0, k)]],
                            add=True)

            @pl.when(2 * t + 2 < nfull)
            def _():
                start_gather(srcI.at[pl.ds(off0 + 2 * k, k)], rowsA, semA)

            wait_gather(srcI.at[pl.ds(off0 + k, k)], rowsB, semB)
            pltpu.sync_copy(rowsB, acc.at[dstI.at[pl.ds(off0 + k, k)]],
                            add=True)
            return carry

        if npair:
            lax.fori_loop(0, npair, pair, 0)
        if odd_full:
            offl = (nfull - 1) * k
            wait_gather(srcI.at[pl.ds(offl, k)], rowsA, semA)
            pltpu.sync_copy(rowsA, acc.at[dstI.at[pl.ds(offl, k)]],
                            add=True)
        if erem:
            offr = nfull * k
            rslice = rowsB.at[pl.ds(0, erem)]
            start_gather(srcI.at[pl.ds(offr, erem)], rslice, semB)
            wait_gather(srcI.at[pl.ds(offr, erem)], rslice, semB)
            pltpu.sync_copy(rslice, acc.at[dstI.at[pl.ds(offr, erem)]],
                            add=True)
        plsc.subcore_barrier()

        @pl.when(cid == 0)
        def _():
            pltpu.sync_copy(acc.at[pl.ds(base_r, rpt)],
                            outa.at[pl.ds(base_r, rpt)])

        @pl.when(cid == 1)
        def _():
            pltpu.sync_copy(acc.at[pl.ds(base_r, rpt)],
                            outb.at[pl.ds(base_r, rpt)])

    return sc_segsum


def _make_sc_segsum_es(n, e, w, k):
    half = e // 2
    ept = half // _NSUB
    nfull = ept // k
    erem = ept % k
    npair = nfull // 2
    odd_full = nfull % 2
    rpt = n // _NSUB
    nz = rpt // k
    zrem = rpt % k
    mesh = plsc.VectorSubcoreMesh(core_axis_name="c", subcore_axis_name="s")

    @functools.partial(
        pl.kernel,
        out_type=[jax.ShapeDtypeStruct((n, w), _F32),
                  jax.ShapeDtypeStruct((n, w), _F32)],
        mesh=mesh,
        scratch_types=[
            pltpu.VMEM((ept,), jnp.int32),
            pltpu.VMEM((ept,), jnp.int32),
            pltpu.VMEM((k, w), _F32),
            pltpu.VMEM((k, w), _F32),
            pltpu.VMEM_SHARED((n, w), _F32),
            pltpu.SemaphoreType.DMA,
            pltpu.SemaphoreType.DMA,
        ],
        compiler_params=pltpu.CompilerParams(use_tc_tiling_on_sc=False),
    )
    def sc_segsum_es(y, ei, outa, outb, srcI, dstI, rowsA, rowsB,
                     acc, semA, semB):
        cid = lax.axis_index("c")
        sid = lax.axis_index("s")

        base_e = cid * half + sid * ept
        pltpu.async_copy(ei.at[0, pl.ds(base_e, ept)], srcI, semA)
        pltpu.async_copy(ei.at[1, pl.ds(base_e, ept)], dstI, semB)

        zero = jnp.zeros((16,), _F32)

        def zrow(i, carry):
            def zcol(j, c2):
                rowsB[i, pl.ds(j * 16, 16)] = zero
                return c2
            return lax.fori_loop(0, w // 16, zcol, carry)

        lax.fori_loop(0, k, zrow, 0)
        base_r = sid * rpt
        for j in range(nz):
            pltpu.sync_copy(rowsB, acc.at[pl.ds(base_r + j * k, k)])
        if zrem:
            pltpu.sync_copy(rowsB.at[pl.ds(0, zrem)],
                            acc.at[pl.ds(base_r + nz * k, zrem)])
        pltpu.make_async_copy(ei.at[0, pl.ds(base_e, ept)], srcI,
                              semA).wait()
        pltpu.make_async_copy(ei.at[1, pl.ds(base_e, ept)], dstI,
                              semB).wait()

        if nfull >= 1:
            pltpu.async_copy(y.at[srcI.at[pl.ds(0, k)]], rowsA, semA)
        plsc.subcore_barrier()

        def pair(t, carry):
            off0 = 2 * t * k
            pltpu.async_copy(y.at[srcI.at[pl.ds(off0 + k, k)]], rowsB,
                             semB)
            pltpu.make_async_copy(y.at[srcI.at[pl.ds(off0, k)]], rowsA,
                                  semA).wait()
            pltpu.sync_copy(rowsA, acc.at[dstI.at[pl.ds(off0, k)]],
                            add=True)

            @pl.when(2 * t + 2 < nfull)
            def _():
                pltpu.async_copy(y.at[srcI.at[pl.ds(off0 + 2 * k, k)]],
                                 rowsA, semA)

            pltpu.make_async_copy(y.at[srcI.at[pl.ds(off0 + k, k)]],
                                  rowsB, semB).wait()
            pltpu.sync_copy(rowsB, acc.at[dstI.at[pl.ds(off0 + k, k)]],
                            add=True)
            return carry

        if npair:
            lax.fori_loop(0, npair, pair, 0)
        if odd_full:
            offl = (nfull - 1) * k
            pltpu.make_async_copy(y.at[srcI.at[pl.ds(offl, k)]], rowsA,
                                  semA).wait()
            pltpu.sync_copy(rowsA, acc.at[dstI.at[pl.ds(offl, k)]],
                            add=True)
        if erem:
            offr = nfull * k
            rslice = rowsB.at[pl.ds(0, erem)]
            pltpu.async_copy(y.at[srcI.at[pl.ds(offr, erem)]], rslice,
                             semB)
            pltpu.make_async_copy(y.at[srcI.at[pl.ds(offr, erem)]],
                                  rslice, semB).wait()
            pltpu.sync_copy(rslice, acc.at[dstI.at[pl.ds(offr, erem)]],
                            add=True)
        plsc.subcore_barrier()

        @pl.when(cid == 0)
        def _():
            pltpu.sync_copy(acc.at[pl.ds(base_r, rpt)],
                            outa.at[pl.ds(base_r, rpt)])

        @pl.when(cid == 1)
        def _():
            pltpu.sync_copy(acc.at[pl.ds(base_r, rpt)],
                            outb.at[pl.ds(base_r, rpt)])

    return sc_segsum_es


def _make_sc_count(n, e, kc):
    half = e // 2
    ept = half // _NSUB
    nchunks = ept // kc
    rpt = n // _NSUB
    nz = rpt // kc
    zrem = rpt % kc
    mesh = plsc.VectorSubcoreMesh(core_axis_name="c", subcore_axis_name="s")

    @functools.partial(
        pl.kernel,
        out_type=[jax.ShapeDtypeStruct((n, 16), _F32),
                  jax.ShapeDtypeStruct((n, 16), _F32)],
        mesh=mesh,
        scratch_types=[
            pltpu.VMEM((kc,), jnp.int32),
            pltpu.VMEM((kc, 16), _F32),
            pltpu.VMEM_SHARED((n, 16), _F32),
        ],
        compiler_params=pltpu.CompilerParams(use_tc_tiling_on_sc=False),
    )
    def sc_count(ei, cnta, cntb, dstb, ones, acc):
        cid = lax.axis_index("c")
        sid = lax.axis_index("s")

        zero = jnp.zeros((16,), _F32)

        def zrow(i, carry):
            ones[i, pl.ds(0, 16)] = zero
            return carry

        lax.fori_loop(0, kc, zrow, 0)
        base_r = sid * rpt
        for j in range(nz):
            pltpu.sync_copy(ones, acc.at[pl.ds(base_r + j * kc, kc)])
        if zrem:
            pltpu.sync_copy(ones.at[pl.ds(0, zrem)],
                            acc.at[pl.ds(base_r + nz * kc, zrem)])

        one = jnp.ones((16,), _F32)

        def orow(i, carry):
            ones[i, pl.ds(0, 16)] = one
            return carry

        lax.fori_loop(0, kc, orow, 0)
        plsc.subcore_barrier()

        base_e = cid * half + sid * ept

        def chunk(i, carry):
            pltpu.sync_copy(ei.at[1, pl.ds(base_e + i * kc, kc)], dstb)
            pltpu.sync_copy(ones, acc.at[dstb], add=True)
            return carry

        lax.fori_loop(0, nchunks, chunk, 0)
        plsc.subcore_barrier()

        @pl.when(cid == 0)
        def _():
            pltpu.sync_copy(acc.at[pl.ds(base_r, rpt)],
                            cnta.at[pl.ds(base_r, rpt)])

        @pl.when(cid == 1)
        def _():
            pltpu.sync_copy(acc.at[pl.ds(base_r, rpt)],
                            cntb.at[pl.ds(base_r, rpt)])

    return sc_count


def _tc1_body(x_ref, wl_ref, ya_ref, yb_ref):
    t = jnp.dot(x_ref[...], wl_ref[...], preferred_element_type=_F32)
    ya_ref[...] = t[:, :128]
    yb_ref[...] = t[:, 128:]


def _tc2_body(sa_ref, sb_ref, ca_ref, cb_ref, x_ref, w1r_ref, b1_ref,
              w2l_ref, w2r_ref, y2_ref, hrc_ref):
    sa = sa_ref[...]
    cnt = jnp.maximum(ca_ref[:, :1] + cb_ref[:, :1], 1.0)
    xr = jnp.dot(x_ref[...], w1r_ref[...], preferred_element_type=_F32)
    h = (jnp.concatenate([sa, sb_ref[...]], axis=1) / cnt
         + xr + b1_ref[...])
    h = jnp.maximum(h, 0.0)
    y2_ref[...] = jnp.dot(h, w2l_ref[...], preferred_element_type=_F32)
    hr = jnp.dot(h, w2r_ref[...], preferred_element_type=_F32)
    zeros = jnp.zeros((sa.shape[0], 15), _F32)
    hrc_ref[...] = jnp.concatenate([hr, cnt, zeros], axis=1)


def _tc3_body(s2a_ref, s2b_ref, hrc_ref, b2_ref, out_ref):
    hrc = hrc_ref[...]
    cnt = hrc[:, 128:129]
    pre = ((s2a_ref[...] + s2b_ref[...]) / cnt
           + hrc[:, :128] + b2_ref[...])
    mx = jnp.max(pre, axis=1, keepdims=True)
    lse = jnp.log(jnp.sum(jnp.exp(pre - mx), axis=1, keepdims=True))
    out_ref[...] = pre - mx - lse


def kernel(x, edge_index, W1_l, W1_r, b1, W2_l, W2_r, b2):
    n, d_in = x.shape
    e = edge_index.shape[1]
    d_h = W1_l.shape[1]
    d_out = W2_l.shape[1]
    rb = 2000
    grid = (n // rb,)
    row_spec = lambda w_: pl.BlockSpec((rb, w_), lambda r: (r, 0))
    full_spec = lambda a, b_: pl.BlockSpec((a, b_), lambda r: (0, 0))

    cnta, cntb = _make_sc_count(n, e, 1000)(edge_index)

    ya, yb = pl.pallas_call(
        _tc1_body,
        grid=grid,
        in_specs=[row_spec(d_in), full_spec(d_in, d_h)],
        out_specs=[row_spec(128), row_spec(128)],
        out_shape=[jax.ShapeDtypeStruct((n, 128), _F32),
                   jax.ShapeDtypeStruct((n, 128), _F32)],
    )(x, W1_l)

    sa, sb = _make_sc_segsum(n, e, 128, 120)(ya, yb, edge_index)

    y2, hrc = pl.pallas_call(
        _tc2_body,
        grid=grid,
        in_specs=[row_spec(128), row_spec(128), row_spec(16), row_spec(16),
                  row_spec(d_in), full_spec(d_in, d_h), full_spec(1, d_h),
                  full_spec(d_h, d_out), full_spec(d_h, d_out)],
        out_specs=[row_spec(128), row_spec(144)],
        out_shape=[jax.ShapeDtypeStruct((n, 128), _F32),
                   jax.ShapeDtypeStruct((n, 144), _F32)],
    )(sa, sb, cnta, cntb, x, W1_r, b1.reshape(1, d_h), W2_l, W2_r)

    s2a, s2b = _make_sc_segsum_es(n, e, 128, 128)(y2, edge_index)

    out = pl.pallas_call(
        _tc3_body,
        grid=grid,
        in_specs=[row_spec(128), row_spec(128), row_spec(144),
                  full_spec(1, d_out)],
        out_specs=row_spec(d_out),
        out_shape=jax.ShapeDtypeStruct((n, d_out), _F32),
    )(s2a, s2b, hrc, b2.reshape(1, d_out))
    return out
